# baseline (speedup 1.0000x reference)
import functools

import jax
import jax.numpy as jnp
from jax import lax
from jax.experimental import pallas as pl
from jax.experimental.pallas import tpu as pltpu
from jax.experimental.pallas import tpu_sc as plsc

NC = 2
NS = 16
NW = NC * NS
L = 16

DP = 128
NBUF = 2

GROUP_STARTS = tuple(range(0, 192, 16)) + (184,)
RUNS = ((0, 128), (128, 72))


def _relayout(table_t):
    C, V = table_t.shape
    n_full = V // 128
    rem = V - n_full * 128
    base_per_w = n_full // NW
    extra_w = n_full - base_per_w * NW

    mesh = plsc.VectorSubcoreMesh(core_axis_name="c", subcore_axis_name="s")

    @functools.partial(
        pl.kernel,
        mesh=mesh,
        out_type=jax.ShapeDtypeStruct((V * C,), jnp.float32),
        scratch_types=[
            pltpu.VMEM((NBUF * 64, 129), jnp.float32),
            pltpu.VMEM((NBUF * 128 * C,), jnp.float32),
            pltpu.SemaphoreType.DMA,
            pltpu.SemaphoreType.DMA,
            pltpu.SemaphoreType.DMA,
            pltpu.SemaphoreType.DMA,
        ],
        compiler_params=pltpu.CompilerParams(
            needs_layout_passes=False, use_tc_tiling_on_sc=True
        ),
    )
    def relayout_kernel(tt_hbm, out_hbm, vin, vout, si0, si1, so0, so1):
        wid = lax.axis_index("s") * NC + lax.axis_index("c")
        n_k = base_per_w + jnp.where(wid < extra_w, 1, 0)
        sem_i = (si0, si1)
        sem_o = (so0, so1)
        lane = lax.iota(jnp.int32, L)

        def blk_of(k):
            return wid + NW * k

        lane_c = lane * C

        def fire_in(k, p):
            blk = blk_of(k)
            for ct in range(8):
                pltpu.async_copy(
                    tt_hbm.at[pl.ds(ct * 8, 8), pl.ds(blk * 128, 128)],
                    vin.at[pl.ds(p * 64 + ct * 8, 8), pl.ds(0, 128)],
                    sem_i[p],
                )

        def wait_in(p):
            for ct in range(8):
                pltpu.make_async_copy(
                    tt_hbm.at[pl.ds(0, 8), pl.ds(0, 128)],
                    vin.at[pl.ds(p * 64 + ct * 8, 8), pl.ds(0, 128)],
                    sem_i[p],
                ).wait()

        def transpose(p, width):
            rowvecs = [p * 64 + cg * L + lane for cg in range(C // L)]
            UNR = 8

            def rbody(r8, carry):
                r0 = r8 * UNR
                for dr in range(UNR):
                    colv = jnp.full((L,), r0 + dr, jnp.int32)
                    for cg in range(C // L):
                        vals = plsc.load_gather(vin, [rowvecs[cg], colv])
                        vout[pl.ds(p * 128 * C + (r0 + dr) * C + cg * L, L)] = vals
                return carry

            lax.fori_loop(0, width // UNR, rbody, 0)

        def fire_out(k, p, width):
            blk = blk_of(k)
            pltpu.async_copy(
                vout.at[pl.ds(p * 128 * C, width * C)],
                out_hbm.at[pl.ds(blk * 128 * C, width * C)],
                sem_o[p],
            )

        def wait_out(p, width):
            pltpu.make_async_copy(
                out_hbm.at[pl.ds(0, width * C)],
                vout.at[pl.ds(p * 128 * C, width * C)],
                sem_o[p],
            ).wait()

        fire_in(0, 0)

        def pair_body(i0, carry):
            k0 = i0 * 2
            k1 = k0 + 1

            fire_in(k1, 1)
            wait_in(0)

            @pl.when(k0 >= 2)
            def _():
                wait_out(0, 128)

            transpose(0, 128)
            fire_out(k0, 0, 128)

            @pl.when(k1 + 1 < n_k)
            def _():
                fire_in(k1 + 1, 0)
            wait_in(1)

            @pl.when(k1 >= 2)
            def _():
                wait_out(1, 128)

            transpose(1, 128)
            fire_out(k1, 1, 128)
            return carry

        lax.fori_loop(0, base_per_w // 2, pair_body, 0)

        @pl.when(n_k > base_per_w)
        def _extra():
            wait_in(0)
            wait_out(0, 128)
            transpose(0, 128)
            fire_out(base_per_w, 0, 128)

        wait_out(0, 128)
        wait_out(1, 128)

        if rem:
            @pl.when(wid == NW - 1)
            def _tail():
                tail_start = wid * 0 + n_full * 128
                for ct in range(8):
                    pltpu.sync_copy(
                        tt_hbm.at[pl.ds(ct * 8, 8), pl.ds(tail_start, 128)],
                        vin.at[pl.ds(ct * 8, 8), pl.ds(0, 128)],
                    )
                transpose(0, rem)
                pltpu.sync_copy(
                    vout.at[pl.ds(0, rem * C)],
                    out_hbm.at[pl.ds(n_full * 128 * C, rem * C)],
                )

    return relayout_kernel(table_t)


def _gather(input_ids, table_flat, B, S, V, D):
    rows_per_w = B // NW
    n_chunks = rows_per_w

    mesh = plsc.VectorSubcoreMesh(core_axis_name="c", subcore_axis_name="s")

    @functools.partial(
        pl.kernel,
        mesh=mesh,
        out_type=jax.ShapeDtypeStruct((B, S, DP), jnp.float32),
        scratch_types=[
            pltpu.VMEM((rows_per_w, S), jnp.int32),
            pltpu.VMEM((NBUF, S, D), jnp.float32),
            pltpu.SemaphoreType.DMA,
            pltpu.SemaphoreType.DMA,
            pltpu.SemaphoreType.DMA,
            pltpu.SemaphoreType.DMA,
        ],
        compiler_params=pltpu.CompilerParams(
            needs_layout_passes=False, use_tc_tiling_on_sc=False
        ),
    )
    def emb_kernel(idx_hbm, table_hbm, out_hbm, idx_v, rows_v, sg0, sg1, so0, so1):
        wid = lax.axis_index("s") * NC + lax.axis_index("c")
        b0w = wid * rows_per_w
        sem_g = (sg0, sg1)
        sem_o = (so0, so1)

        pltpu.sync_copy(idx_hbm.at[pl.ds(b0w, rows_per_w)], idx_v)

        def start_gather(c, b):
            for off, n in RUNS:
                pltpu.async_copy(
                    table_hbm.at[idx_v.at[c, pl.ds(off, n)]],
                    rows_v.at[b].at[pl.ds(off, n)],
                    sem_g[b],
                )

        def wait_gather(b):
            for off, n in RUNS:
                pltpu.make_async_copy(
                    table_hbm.at[pl.ds(0, n)],
                    rows_v.at[b].at[pl.ds(off, n)],
                    sem_g[b],
                ).wait()

        def start_out(c, b):
            pltpu.async_copy(
                rows_v.at[b],
                out_hbm.at[b0w + c].at[:, pl.ds(0, D)],
                sem_o[b],
            )

        def wait_out(b):
            pltpu.make_async_copy(
                table_hbm.at[pl.ds(0, S)],
                rows_v.at[b],
                sem_o[b],
            ).wait()

        def fixup(c, b):
            m = None
            for off in GROUP_STARTS:
                iv = idx_v[c, pl.ds(off, L)]
                m = iv if m is None else jnp.minimum(m, iv)
            pad_cnt = plsc.all_reduce_population_count(m == 0)

            @pl.when(pad_cnt[0] != 0)
            def _fixup():
                zeros = jnp.zeros((L,), jnp.float32)
                lane = lax.iota(jnp.int32, L)

                def group_body(g, carry2):
                    off = jnp.minimum(g * L, S - L)
                    iv = idx_v[c, pl.ds(off, L)]
                    is_pad = iv == 0
                    gcnt = plsc.all_reduce_population_count(is_pad)

                    @pl.when(gcnt[0] != 0)
                    def _zero_rows():
                        srow = off + lane
                        for col in range(D):
                            plsc.store_scatter(
                                rows_v.at[b],
                                [srow, jnp.full((L,), col, jnp.int32)],
                                zeros,
                                mask=is_pad,
                            )

                    return carry2

                lax.fori_loop(0, len(GROUP_STARTS), group_body, 0)

        start_gather(0, 0)

        def pair_body(i0, carry):
            c0 = i0 * 2
            c1 = c0 + 1

            @pl.when(c0 > 0)
            def _():
                wait_out(1)
            start_gather(c1, 1)
            wait_gather(0)
            fixup(c0, 0)
            start_out(c0, 0)

            @pl.when(c1 < n_chunks - 1)
            def _():
                wait_out(0)
                start_gather(c1 + 1, 0)
            wait_gather(1)
            fixup(c1, 1)
            start_out(c1, 1)
            return carry

        lax.fori_loop(0, n_chunks // 2, pair_body, 0)
        wait_out(0)
        wait_out(1)

    return emb_kernel(input_ids, table_flat)


def kernel(input_ids, table):
    B, S = input_ids.shape
    V, D = table.shape
    t2 = jnp.concatenate([table[0::2, :], table[1::2, :]], axis=1)
    table_lin = t2.reshape(V, D)

    out = _gather(input_ids, table_lin, B, S, V, D)
    return out[:, :, :D]


# speedup vs baseline: 10.5212x; 10.5212x over previous
import functools

import jax
import jax.numpy as jnp
from jax import lax
from jax.experimental import pallas as pl
from jax.experimental.pallas import tpu as pltpu
from jax.experimental.pallas import tpu_sc as plsc

NC = 2
NS = 16
NW = NC * NS
L = 16

DP = 128
NBUF = 2

GROUP_STARTS = tuple(range(0, 192, 16)) + (184,)
RUNS = ((0, 128), (128, 72))


def kernel(input_ids, table):
    B, S = input_ids.shape
    V, D = table.shape
    rows_per_w = B // NW
    n_chunks = rows_per_w

    t2 = jnp.pad(table, ((0, 0), (0, DP - D))).reshape(2 * V, D)

    mesh = plsc.VectorSubcoreMesh(core_axis_name="c", subcore_axis_name="s")

    @functools.partial(
        pl.kernel,
        mesh=mesh,
        out_type=jax.ShapeDtypeStruct((B, S, DP), jnp.float32),
        scratch_types=[
            pltpu.VMEM((rows_per_w, S), jnp.int32),
            pltpu.VMEM((NBUF, S), jnp.int32),
            pltpu.VMEM((NBUF, S, D), jnp.float32),
            pltpu.SemaphoreType.DMA,
            pltpu.SemaphoreType.DMA,
            pltpu.SemaphoreType.DMA,
            pltpu.SemaphoreType.DMA,
        ],
        compiler_params=pltpu.CompilerParams(
            needs_layout_passes=False, use_tc_tiling_on_sc=False
        ),
    )
    def emb_kernel(idx_hbm, table_hbm, out_hbm, idx_v, idx2_v, rows_v,
                   sg0, sg1, so0, so1):
        wid = lax.axis_index("s") * NC + lax.axis_index("c")
        b0w = wid * rows_per_w
        sem_g = (sg0, sg1)
        sem_o = (so0, so1)

        pltpu.sync_copy(idx_hbm.at[pl.ds(b0w, rows_per_w)], idx_v)

        def start_gather(c, b):
            for off in GROUP_STARTS:
                iv = idx_v[c, pl.ds(off, L)]
                idx2_v[b, pl.ds(off, L)] = iv + iv
            for off, n in RUNS:
                pltpu.async_copy(
                    table_hbm.at[idx2_v.at[b, pl.ds(off, n)]],
                    rows_v.at[b].at[pl.ds(off, n)],
                    sem_g[b],
                )

        def wait_gather(b):
            for off, n in RUNS:
                pltpu.make_async_copy(
                    table_hbm.at[pl.ds(0, n)],
                    rows_v.at[b].at[pl.ds(off, n)],
                    sem_g[b],
                ).wait()

        def start_out(c, b):
            pltpu.async_copy(
                rows_v.at[b],
                out_hbm.at[b0w + c].at[:, pl.ds(0, D)],
                sem_o[b],
            )

        def wait_out(b):
            pltpu.make_async_copy(
                table_hbm.at[pl.ds(0, S)],
                rows_v.at[b],
                sem_o[b],
            ).wait()

        def fixup(c, b):
            m = None
            for off in GROUP_STARTS:
                iv = idx_v[c, pl.ds(off, L)]
                m = iv if m is None else jnp.minimum(m, iv)
            pad_cnt = plsc.all_reduce_population_count(m == 0)

            @pl.when(pad_cnt[0] != 0)
            def _fixup():
                zeros = jnp.zeros((L,), jnp.float32)
                lane = lax.iota(jnp.int32, L)

                def group_body(g, carry2):
                    off = jnp.minimum(g * L, S - L)
                    iv = idx_v[c, pl.ds(off, L)]
                    is_pad = iv == 0
                    gcnt = plsc.all_reduce_population_count(is_pad)

                    @pl.when(gcnt[0] != 0)
                    def _zero_rows():
                        srow = off + lane
                        for col in range(D):
                            plsc.store_scatter(
                                rows_v.at[b],
                                [srow, jnp.full((L,), col, jnp.int32)],
                                zeros,
                                mask=is_pad,
                            )

                    return carry2

                lax.fori_loop(0, len(GROUP_STARTS), group_body, 0)

        start_gather(0, 0)

        def pair_body(i0, carry):
            c0 = i0 * 2
            c1 = c0 + 1

            @pl.when(c0 > 0)
            def _():
                wait_out(1)
            start_gather(c1, 1)
            wait_gather(0)
            fixup(c0, 0)
            start_out(c0, 0)

            @pl.when(c1 < n_chunks - 1)
            def _():
                wait_out(0)
                start_gather(c1 + 1, 0)
            wait_gather(1)
            fixup(c1, 1)
            start_out(c1, 1)
            return carry

        lax.fori_loop(0, n_chunks // 2, pair_body, 0)
        wait_out(0)
        wait_out(1)

    return emb_kernel(input_ids, t2)[:, :, :D]


# speedup vs baseline: 10.5541x; 1.0031x over previous
import functools

import jax
import jax.numpy as jnp
from jax import lax
from jax.experimental import pallas as pl
from jax.experimental.pallas import tpu as pltpu
from jax.experimental.pallas import tpu_sc as plsc

NC = 2
NS = 16
NW = NC * NS
L = 16

DP = 128
NBUF = 2
RPC = 2

GROUP_STARTS = tuple(range(0, 192, 16)) + (184,)
RUNS = ((0, 128), (128, 72))


def kernel(input_ids, table):
    B, S = input_ids.shape
    V, D = table.shape
    rows_per_w = B // NW
    n_chunks = rows_per_w // RPC

    t2 = jnp.pad(table, ((0, 0), (0, DP - D))).reshape(2 * V, D)

    mesh = plsc.VectorSubcoreMesh(core_axis_name="c", subcore_axis_name="s")

    @functools.partial(
        pl.kernel,
        mesh=mesh,
        out_type=jax.ShapeDtypeStruct((B, S, DP), jnp.float32),
        scratch_types=[
            pltpu.VMEM((rows_per_w, S), jnp.int32),
            pltpu.VMEM((NBUF, RPC, S), jnp.int32),
            pltpu.VMEM((NBUF, RPC, S, D), jnp.float32),
            pltpu.SemaphoreType.DMA,
            pltpu.SemaphoreType.DMA,
            pltpu.SemaphoreType.DMA,
            pltpu.SemaphoreType.DMA,
        ],
        compiler_params=pltpu.CompilerParams(
            needs_layout_passes=False, use_tc_tiling_on_sc=False
        ),
    )
    def emb_kernel(idx_hbm, table_hbm, out_hbm, idx_v, idx2_v, rows_v,
                   sg0, sg1, so0, so1):
        wid = lax.axis_index("s") * NC + lax.axis_index("c")
        b0w = wid * rows_per_w
        sem_g = (sg0, sg1)
        sem_o = (so0, so1)

        pltpu.sync_copy(idx_hbm.at[pl.ds(b0w, rows_per_w)], idx_v)

        def start_gather(c, b):
            for r in range(RPC):
                for off in GROUP_STARTS:
                    iv = idx_v[c * RPC + r, pl.ds(off, L)]
                    idx2_v[b, r, pl.ds(off, L)] = iv + iv
                for off, n in RUNS:
                    pltpu.async_copy(
                        table_hbm.at[idx2_v.at[b, r, pl.ds(off, n)]],
                        rows_v.at[b, r].at[pl.ds(off, n)],
                        sem_g[b],
                    )

        def wait_gather(b):
            for r in range(RPC):
                for off, n in RUNS:
                    pltpu.make_async_copy(
                        table_hbm.at[pl.ds(0, n)],
                        rows_v.at[b, r].at[pl.ds(off, n)],
                        sem_g[b],
                    ).wait()

        def start_out(c, b):
            pltpu.async_copy(
                rows_v.at[b],
                out_hbm.at[pl.ds(b0w + c * RPC, RPC)].at[:, :, pl.ds(0, D)],
                sem_o[b],
            )

        def wait_out(b):
            pltpu.make_async_copy(
                table_hbm.at[pl.ds(0, RPC * S)],
                rows_v.at[b],
                sem_o[b],
            ).wait()

        def fixup(c, b):
            m = None
            for r in range(RPC):
                for off in GROUP_STARTS:
                    iv = idx_v[c * RPC + r, pl.ds(off, L)]
                    m = iv if m is None else jnp.minimum(m, iv)
            pad_cnt = plsc.all_reduce_population_count(m == 0)

            @pl.when(pad_cnt[0] != 0)
            def _fixup():
                zeros = jnp.zeros((L,), jnp.float32)
                lane = lax.iota(jnp.int32, L)

                for r in range(RPC):
                    def group_body(g, carry2, r=r):
                        off = jnp.minimum(g * L, S - L)
                        iv = idx_v[c * RPC + r, pl.ds(off, L)]
                        is_pad = iv == 0
                        gcnt = plsc.all_reduce_population_count(is_pad)

                        @pl.when(gcnt[0] != 0)
                        def _zero_rows():
                            srow = off + lane
                            for col in range(D):
                                plsc.store_scatter(
                                    rows_v.at[b, r],
                                    [srow, jnp.full((L,), col, jnp.int32)],
                                    zeros,
                                    mask=is_pad,
                                )

                        return carry2

                    lax.fori_loop(0, len(GROUP_STARTS), group_body, 0)

        start_gather(0, 0)

        def pair_body(i0, carry):
            c0 = i0 * 2
            c1 = c0 + 1

            @pl.when(c0 > 0)
            def _():
                wait_out(1)
            start_gather(c1, 1)
            wait_gather(0)
            fixup(c0, 0)
            start_out(c0, 0)

            @pl.when(c1 < n_chunks - 1)
            def _():
                wait_out(0)
                start_gather(c1 + 1, 0)
            wait_gather(1)
            fixup(c1, 1)
            start_out(c1, 1)
            return carry

        lax.fori_loop(0, n_chunks // 2, pair_body, 0)
        wait_out(0)
        wait_out(1)

    return emb_kernel(input_ids, t2)[:, :, :D]
